# Initial kernel scaffold; baseline (speedup 1.0000x reference)
import functools

import jax
import jax.numpy as jnp
from jax import lax
from jax.experimental import pallas as pl
from jax.experimental.pallas import tpu as pltpu
from jax.experimental.pallas import tpu_sc as plsc

N_NODES = 10000
N_EDGES = 320000
DIM = 128

NC = 2
NS = 16
NW = NC * NS
EPW = N_EDGES // NW
CH = 125
NCHUNK = EPW // CH
RPT = N_NODES // NS
ZR = 125
LANES = 16


def _sc_aggregate(x, src3, dst3):
    mesh = plsc.VectorSubcoreMesh(core_axis_name="c", subcore_axis_name="s")

    @functools.partial(
        pl.kernel,
        mesh=mesh,
        out_type=jax.ShapeDtypeStruct((NC, N_NODES, DIM), jnp.float32),
        scratch_types=[
            pltpu.VMEM((NCHUNK, CH), jnp.int32),
            pltpu.VMEM((NCHUNK, CH), jnp.int32),
            pltpu.VMEM((2, CH, DIM), jnp.float32),
            pltpu.VMEM((ZR, DIM), jnp.float32),
            pltpu.VMEM_SHARED((N_NODES, DIM), jnp.float32),
            pltpu.SemaphoreType.DMA,
        ],
    )
    def k(x_hbm, src_hbm, dst_hbm, out_hbm, sidx, didx, rows, zbuf, acc, gsem):
        c = lax.axis_index("c")
        s = lax.axis_index("s")
        wid = s * NC + c

        def zbody(t, _):
            i = t // (DIM // LANES)
            j = t % (DIM // LANES)
            zbuf[i, pl.ds(j * LANES, LANES)] = jnp.zeros((LANES,), jnp.float32)
            return 0
        lax.fori_loop(0, ZR * (DIM // LANES), zbody, 0)
        for j in range(RPT // ZR):
            pltpu.sync_copy(zbuf, acc.at[pl.ds(s * RPT + j * ZR, ZR)])
        plsc.subcore_barrier()

        pltpu.sync_copy(src_hbm.at[wid], sidx)
        pltpu.sync_copy(dst_hbm.at[wid], didx)

        pltpu.make_async_copy(x_hbm.at[sidx.at[0]], rows.at[0], gsem).start()

        def chunk_body(i0, _):
            for b in range(2):
                i = i0 * 2 + b
                nb = (b + 1) % 2

                @pl.when(i + 1 < NCHUNK)
                def _start_next():
                    pltpu.make_async_copy(
                        x_hbm.at[sidx.at[i + 1]], rows.at[nb], gsem).start()

                pltpu.make_async_copy(
                    x_hbm.at[sidx.at[i]], rows.at[b], gsem).wait()
                pltpu.sync_copy(rows.at[b], acc.at[didx.at[i]], add=True)
            return 0
        lax.fori_loop(0, NCHUNK // 2, chunk_body, 0)

        plsc.subcore_barrier()
        pltpu.sync_copy(acc.at[pl.ds(s * RPT, RPT)],
                        out_hbm.at[c, pl.ds(s * RPT, RPT)])

    return k(x, src3, dst3)


def _tc_combine(x, part, W1t, W2t):
    BR = 1000
    grid = N_NODES // BR

    def body(x_ref, p0_ref, p1_ref, w1_ref, w2_ref, o_ref):
        agg = p0_ref[...] + p1_ref[...]
        acc = jnp.dot(x_ref[...], w1_ref[...],
                      preferred_element_type=jnp.float32)
        acc += jnp.dot(agg, w2_ref[...], preferred_element_type=jnp.float32)
        o_ref[...] = jnp.maximum(acc, 0.0)

    return pl.pallas_call(
        body,
        grid=(grid,),
        in_specs=[
            pl.BlockSpec((BR, DIM), lambda i: (i, 0)),
            pl.BlockSpec((BR, DIM), lambda i: (i, 0)),
            pl.BlockSpec((BR, DIM), lambda i: (i, 0)),
            pl.BlockSpec((DIM, DIM), lambda i: (0, 0)),
            pl.BlockSpec((DIM, DIM), lambda i: (0, 0)),
        ],
        out_specs=pl.BlockSpec((BR, DIM), lambda i: (i, 0)),
        out_shape=jax.ShapeDtypeStruct((N_NODES, DIM), jnp.float32),
    )(x, part[0], part[1], W1t, W2t)


def kernel(x, edge_index, W1, W2):
    src = edge_index[0].astype(jnp.int32).reshape(NW, NCHUNK, CH)
    dst = edge_index[1].astype(jnp.int32).reshape(NW, NCHUNK, CH)
    part = _sc_aggregate(x, src, dst)
    return _tc_combine(x, part, W1.T, W2.T)


# speedup vs baseline: 9.0694x; 9.0694x over previous
import functools

import jax
import jax.numpy as jnp
from jax import lax
from jax.experimental import pallas as pl
from jax.experimental.pallas import tpu as pltpu
from jax.experimental.pallas import tpu_sc as plsc

N_NODES = 10000
N_EDGES = 320000
DIM = 128
HD = DIM // 2

NC = 2
NS = 16
EPT = N_EDGES // NS
CH = 80
NCHUNK = EPT // CH
N_PAD = 10240
RPT = N_PAD // NS
LANES = 16


def _sc_aggregate(x2, src3, dst3):
    mesh = plsc.VectorSubcoreMesh(core_axis_name="c", subcore_axis_name="s")

    @functools.partial(
        pl.kernel,
        mesh=mesh,
        out_type=jax.ShapeDtypeStruct((NC, N_PAD, HD), jnp.float32),
        compiler_params=pltpu.CompilerParams(use_tc_tiling_on_sc=False),
        scratch_types=[
            pltpu.VMEM((NCHUNK, CH), jnp.int32),
            pltpu.VMEM((NCHUNK, CH), jnp.int32),
            pltpu.VMEM((2, CH, HD), jnp.float32),
            pltpu.VMEM_SHARED((N_PAD, HD), jnp.float32),
            pltpu.SemaphoreType.DMA,
        ],
    )
    def k(x_hbm, src_hbm, dst_hbm, out_hbm, sidx, didx, rows, acc, gsem):
        c = lax.axis_index("c")
        s = lax.axis_index("s")

        def zbody(t, _):
            i = t // (HD // LANES)
            j = t % (HD // LANES)
            rows[0, i, pl.ds(j * LANES, LANES)] = jnp.zeros((LANES,),
                                                            jnp.float32)
            return 0
        lax.fori_loop(0, CH * (HD // LANES), zbody, 0)
        for j in range(RPT // CH):
            pltpu.sync_copy(rows.at[0],
                            acc.at[pl.ds(s * RPT + j * CH, CH)])

        pltpu.sync_copy(src_hbm.at[s], sidx)
        pltpu.sync_copy(dst_hbm.at[s], didx)

        def ibody(t, _):
            i = t // (CH // LANES)
            j = t % (CH // LANES)
            sl = pl.ds(j * LANES, LANES)
            sidx[i, sl] = sidx[i, sl] + c
            return 0
        lax.fori_loop(0, NCHUNK * (CH // LANES), ibody, 0)
        plsc.subcore_barrier()

        pltpu.make_async_copy(x_hbm.at[sidx.at[0]], rows.at[0], gsem).start()

        def chunk_body(i, _):
            b = lax.rem(i, 2)
            nb = 1 - b

            @pl.when(i + 1 < NCHUNK)
            def _start_next():
                pltpu.make_async_copy(
                    x_hbm.at[sidx.at[i + 1]], rows.at[nb], gsem).start()

            pltpu.make_async_copy(
                x_hbm.at[sidx.at[i]], rows.at[b], gsem).wait()
            pltpu.sync_copy(rows.at[b], acc.at[didx.at[i]], add=True)
            return 0
        lax.fori_loop(0, NCHUNK, chunk_body, 0)

        plsc.subcore_barrier()
        pltpu.sync_copy(acc.at[pl.ds(s * RPT, RPT)],
                        out_hbm.at[c, pl.ds(s * RPT, RPT)])

    return k(x2, src3, dst3)


def _tc_combine(x, p0, p1, W1t, W2tA, W2tB):
    BR = 1000
    grid = N_NODES // BR

    def body(x_ref, p0_ref, p1_ref, w1_ref, w2a_ref, w2b_ref, o_ref):
        acc = jnp.dot(x_ref[...], w1_ref[...],
                      preferred_element_type=jnp.float32)
        acc += jnp.dot(p0_ref[...], w2a_ref[...],
                       preferred_element_type=jnp.float32)
        acc += jnp.dot(p1_ref[...], w2b_ref[...],
                       preferred_element_type=jnp.float32)
        o_ref[...] = jnp.maximum(acc, 0.0)

    return pl.pallas_call(
        body,
        grid=(grid,),
        in_specs=[
            pl.BlockSpec((BR, DIM), lambda i: (i, 0)),
            pl.BlockSpec((BR, HD), lambda i: (i, 0)),
            pl.BlockSpec((BR, HD), lambda i: (i, 0)),
            pl.BlockSpec((DIM, DIM), lambda i: (0, 0)),
            pl.BlockSpec((HD, DIM), lambda i: (0, 0)),
            pl.BlockSpec((HD, DIM), lambda i: (0, 0)),
        ],
        out_specs=pl.BlockSpec((BR, DIM), lambda i: (i, 0)),
        out_shape=jax.ShapeDtypeStruct((N_NODES, DIM), jnp.float32),
    )(x, p0, p1, W1t, W2tA, W2tB)


def kernel(x, edge_index, W1, W2):
    src = edge_index[0].astype(jnp.int32)
    dst = edge_index[1].astype(jnp.int32)
    src3 = (2 * src).reshape(NS, NCHUNK, CH)
    dst3 = dst.reshape(NS, NCHUNK, CH)
    x2 = x.reshape(2 * N_NODES, HD)
    part = _sc_aggregate(x2, src3, dst3)
    p0 = part[0, :N_NODES, :]
    p1 = part[1, :N_NODES, :]
    W2t = W2.T
    return _tc_combine(x, p0, p1, W1.T, W2t[:HD], W2t[HD:])


# speedup vs baseline: 12.9012x; 1.4225x over previous
import functools

import jax
import jax.numpy as jnp
from jax import lax
from jax.experimental import pallas as pl
from jax.experimental.pallas import tpu as pltpu
from jax.experimental.pallas import tpu_sc as plsc

N_NODES = 10000
N_EDGES = 320000
DIM = 128
HD = DIM // 2

NC = 2
NS = 16
EPT = N_EDGES // NS
CH = 125
NCHUNK = EPT // CH
NBUF = 4
N_PAD = 10240
RPT = N_PAD // NS
ZCH = 120
LANES = 16


def _sc_aggregate(x2, src4, dst3):
    mesh = plsc.VectorSubcoreMesh(core_axis_name="c", subcore_axis_name="s")

    @functools.partial(
        pl.kernel,
        mesh=mesh,
        out_type=jax.ShapeDtypeStruct((NC, N_PAD, HD), jnp.float32),
        compiler_params=pltpu.CompilerParams(use_tc_tiling_on_sc=False),
        scratch_types=[
            pltpu.VMEM((NCHUNK, CH), jnp.int32),
            pltpu.VMEM((NCHUNK, CH), jnp.int32),
            pltpu.VMEM((NBUF, CH, HD), jnp.float32),
            pltpu.VMEM_SHARED((N_PAD, HD), jnp.float32),
            pltpu.SemaphoreType.DMA,
            pltpu.SemaphoreType.DMA,
        ],
    )
    def k(x_hbm, src_hbm, dst_hbm, out_hbm, sidx, didx, rows, acc, gsem, ssem):
        c = lax.axis_index("c")
        s = lax.axis_index("s")

        def zbody(t, _):
            i = t // (HD // LANES)
            j = t % (HD // LANES)
            rows[0, i, pl.ds(j * LANES, LANES)] = jnp.zeros((LANES,),
                                                            jnp.float32)
            return 0
        lax.fori_loop(0, ZCH * (HD // LANES), zbody, 0)
        for j in range(RPT // ZCH + 1):
            rr = min(ZCH, RPT - j * ZCH)
            pltpu.sync_copy(rows.at[0, pl.ds(0, rr)],
                            acc.at[pl.ds(s * RPT + j * ZCH, rr)])

        pltpu.sync_copy(src_hbm.at[c, s], sidx)
        pltpu.sync_copy(dst_hbm.at[s], didx)
        plsc.subcore_barrier()

        for p in range(NBUF - 1):
            pltpu.async_copy(x_hbm.at[sidx.at[p]], rows.at[p], gsem)

        def chunk_body(i, _):
            b = lax.rem(i, NBUF)
            pltpu.make_async_copy(
                x_hbm.at[sidx.at[i]], rows.at[b], gsem).wait()
            pltpu.async_copy(rows.at[b], acc.at[didx.at[i]], ssem, add=True)

            nxt = i + NBUF - 1
            nb = lax.rem(nxt, NBUF)

            @pl.when(nxt < NCHUNK)
            def _prefetch():
                @pl.when(i >= 1)
                def _drain_one():
                    pltpu.make_async_copy(
                        rows.at[nb], acc.at[didx.at[i]], ssem).wait()
                pltpu.async_copy(x_hbm.at[sidx.at[nxt]], rows.at[nb], gsem)
            return 0
        lax.fori_loop(0, NCHUNK, chunk_body, 0)

        for p in range(NBUF):
            pltpu.make_async_copy(
                rows.at[p], acc.at[didx.at[0]], ssem).wait()

        plsc.subcore_barrier()
        pltpu.sync_copy(acc.at[pl.ds(s * RPT, RPT)],
                        out_hbm.at[c, pl.ds(s * RPT, RPT)])

    return k(x2, src4, dst3)


def _tc_combine(x, part, W1t, W2tA, W2tB):
    BR = 1000
    grid = N_NODES // BR

    def body(x_ref, p0_ref, p1_ref, w1_ref, w2a_ref, w2b_ref, o_ref):
        acc = jnp.dot(x_ref[...], w1_ref[...],
                      preferred_element_type=jnp.float32)
        acc += jnp.dot(p0_ref[0], w2a_ref[...],
                       preferred_element_type=jnp.float32)
        acc += jnp.dot(p1_ref[0], w2b_ref[...],
                       preferred_element_type=jnp.float32)
        o_ref[...] = jnp.maximum(acc, 0.0)

    return pl.pallas_call(
        body,
        grid=(grid,),
        in_specs=[
            pl.BlockSpec((BR, DIM), lambda i: (i, 0)),
            pl.BlockSpec((1, BR, HD), lambda i: (0, i, 0)),
            pl.BlockSpec((1, BR, HD), lambda i: (1, i, 0)),
            pl.BlockSpec((DIM, DIM), lambda i: (0, 0)),
            pl.BlockSpec((HD, DIM), lambda i: (0, 0)),
            pl.BlockSpec((HD, DIM), lambda i: (0, 0)),
        ],
        out_specs=pl.BlockSpec((BR, DIM), lambda i: (i, 0)),
        out_shape=jax.ShapeDtypeStruct((N_NODES, DIM), jnp.float32),
    )(x, part, part, W1t, W2tA, W2tB)


def kernel(x, edge_index, W1, W2):
    src = edge_index[0].astype(jnp.int32)
    dst = edge_index[1].astype(jnp.int32)
    src2 = 2 * src
    src4 = jnp.stack([src2, src2 + 1]).reshape(NC, NS, NCHUNK, CH)
    dst3 = dst.reshape(NS, NCHUNK, CH)
    x2 = x.reshape(2 * N_NODES, HD)
    part = _sc_aggregate(x2, src4, dst3)
    W2t = W2.T
    return _tc_combine(x, part, W1.T, W2t[:HD], W2t[HD:])
